# Initial kernel scaffold; baseline (speedup 1.0000x reference)
import jax
import jax.numpy as jnp
from jax.experimental import pallas as pl


def kernel(X_idxs, W):
    raise NotImplementedError("write your pallas kernel here")


# speedup vs baseline: 4.2750x; 4.2750x over previous
import jax
import jax.numpy as jnp
from jax import lax
from jax.experimental import pallas as pl
from jax.experimental.pallas import tpu as pltpu
from jax.experimental.pallas import tpu_sc as plsc

D = 128
B = 16384
NC = 2
NS = 16
L = 16
NW = NC * NS
BPW = B // NW
CB = 128
NCHUNK = BPW // CB
NG = CB // L


def _body(idx0_hbm, idx1_hbm, w_hbm, out_hbm,
          idx0c, idx1c, rows0, rows1, out_v, sem0, sem1):
    wid = lax.axis_index("s") * NC + lax.axis_index("c")
    base = wid * BPW

    def chunk(c, carry):
        cbase = base + c * CB
        pltpu.sync_copy(idx0_hbm.at[pl.ds(cbase, CB)], idx0c)
        pltpu.sync_copy(idx1_hbm.at[pl.ds(cbase, CB)], idx1c)
        cp0 = pltpu.async_copy(w_hbm.at[idx0c], rows0, sem0)
        cp1 = pltpu.async_copy(w_hbm.at[idx1c], rows1, sem1)
        cp0.wait()
        cp1.wait()

        lanes = lax.iota(jnp.int32, L)

        def group(g, carry2):
            ebase = g * L
            vec = jnp.zeros((L,), jnp.float32)
            for l in range(L):
                e = ebase + l
                acc = rows0[e, pl.ds(0, L)] * rows1[e, pl.ds(0, L)]
                for s in range(1, D // L):
                    acc = acc + (rows0[e, pl.ds(s * L, L)]
                                 * rows1[e, pl.ds(s * L, L)])
                red = jnp.sum(acc)
                vec = jnp.where(lanes == l, red, vec)
            out_v[pl.ds(c * CB + g * L, L)] = vec
            return carry2

        lax.fori_loop(0, NG, group, 0)
        return carry

    lax.fori_loop(0, NCHUNK, chunk, 0)
    pltpu.sync_copy(out_v, out_hbm.at[pl.ds(base, BPW)])


def kernel(X_idxs, W):
    idx0 = X_idxs[:, 0].astype(jnp.int32)
    idx1 = X_idxs[:, 1].astype(jnp.int32)
    mesh = plsc.VectorSubcoreMesh(core_axis_name="c", subcore_axis_name="s")
    f = pl.kernel(
        _body,
        out_type=jax.ShapeDtypeStruct((B,), jnp.float32),
        mesh=mesh,
        compiler_params=pltpu.CompilerParams(needs_layout_passes=False),
        scratch_types=[
            pltpu.VMEM((CB,), jnp.int32),
            pltpu.VMEM((CB,), jnp.int32),
            pltpu.VMEM((CB, D), jnp.float32),
            pltpu.VMEM((CB, D), jnp.float32),
            pltpu.VMEM((BPW,), jnp.float32),
            pltpu.SemaphoreType.DMA,
            pltpu.SemaphoreType.DMA,
        ],
    )
    return f(idx0, idx1, W)
